# baseline (speedup 1.0000x reference)
import functools

import jax
import jax.numpy as jnp
from jax import lax
from jax.experimental import pallas as pl
from jax.experimental.pallas import tpu as pltpu
from jax.experimental.pallas import tpu_sc as plsc

_N_EMB = 512
_DIM = 32
_BLK = 8192
_N_TOK = 8192

_NC = 2
_NS = 16
_NW = _NC * _NS
_ROWS_PER_W = _N_TOK // _NW
_IDX_CHUNK = 128
_CHUNKS = _ROWS_PER_W // _IDX_CHUNK


def _tc_body(w_ref, e_ref, idx_ref, p_ref):
    w = w_ref[...]
    e = e_ref[...]
    e2 = jnp.sum(e * e, axis=1, keepdims=True)
    e_aug = jnp.concatenate([e, e2], axis=1)
    ones = jnp.ones((w.shape[0], 1), jnp.float32)
    w_aug = jnp.concatenate([-2.0 * w, ones], axis=1)
    scores = jax.lax.dot_general(
        e_aug, w_aug, (((1,), (1,)), ((), ())),
        preferred_element_type=jnp.float32,
        precision=jax.lax.Precision.HIGHEST)
    m = jnp.min(scores, axis=0, keepdims=True)
    ids = jax.lax.broadcasted_iota(jnp.int32, scores.shape, 0)
    idxv = jnp.min(
        jnp.where(scores == m, ids, _N_EMB), axis=0, keepdims=True)
    idx_ref[...] = idxv.reshape(1, 1, _BLK)
    part = jnp.sum(m) + jnp.sum(w * w)
    p_ref[0, 0] = part * (1.25 / (_N_TOK * _DIM))


def _tc_argmin(flat, embeddings):
    nblk = _N_TOK // _BLK
    return pl.pallas_call(
        _tc_body,
        grid=(nblk,),
        in_specs=[
            pl.BlockSpec((_BLK, _DIM), lambda i: (i, 0)),
            pl.BlockSpec((_N_EMB, _DIM), lambda i: (0, 0)),
        ],
        out_specs=[
            pl.BlockSpec((1, 1, _BLK), lambda i: (i, 0, 0)),
            pl.BlockSpec(memory_space=pltpu.SMEM),
        ],
        out_shape=[
            jax.ShapeDtypeStruct((_N_TOK // _BLK, 1, _BLK), jnp.int32),
            jax.ShapeDtypeStruct((1, 1), jnp.float32),
        ],
    )(flat, embeddings)


@functools.partial(
    pl.kernel,
    mesh=plsc.VectorSubcoreMesh(core_axis_name="c", subcore_axis_name="s"),
    out_type=jax.ShapeDtypeStruct((_N_TOK, _DIM), jnp.float32),
    scratch_types=[
        pltpu.VMEM((_CHUNKS, _IDX_CHUNK), jnp.int32),
        pltpu.VMEM((_ROWS_PER_W, _DIM), jnp.float32),
        pltpu.SemaphoreType.DMA,
    ],
    compiler_params=pltpu.CompilerParams(
        use_tc_tiling_on_sc=False, skip_device_barrier=True),
)
def _sc_gather(table_hbm, idx_hbm, out_hbm, idx_v, rows_v, sem):
    wid = lax.axis_index("s") * _NC + lax.axis_index("c")
    base = wid * _ROWS_PER_W
    pltpu.sync_copy(idx_hbm.at[pl.ds(wid * _CHUNKS, _CHUNKS)], idx_v)
    copies = []
    for j in range(_CHUNKS):
        copies.append(pltpu.async_copy(
            table_hbm.at[idx_v.at[j]],
            rows_v.at[pl.ds(j * _IDX_CHUNK, _IDX_CHUNK)], sem))
    for c in copies:
        c.wait()
    pltpu.sync_copy(rows_v, out_hbm.at[pl.ds(base, _ROWS_PER_W)])


def kernel(weights, embeddings):
    orig_shape = weights.shape
    flat = weights.reshape(-1, _DIM)
    idx, loss = _tc_argmin(flat, embeddings)
    q = _sc_gather(embeddings, idx.reshape(_N_TOK // _IDX_CHUNK, _IDX_CHUNK))
    return q.reshape(orig_shape), loss.reshape(())


# speedup vs baseline: 1.0010x; 1.0010x over previous
import functools

import jax
import jax.numpy as jnp
from jax import lax
from jax.experimental import pallas as pl
from jax.experimental.pallas import tpu as pltpu
from jax.experimental.pallas import tpu_sc as plsc

_N_EMB = 512
_DIM = 32
_BLK = 8192
_N_TOK = 8192

_NC = 2
_NS = 16
_NW = _NC * _NS
_ROWS_PER_W = _N_TOK // _NW
_IDX_CHUNK = 128
_CHUNKS = _ROWS_PER_W // _IDX_CHUNK


def _tc_body(w_ref, e_ref, idx_ref, p_ref):
    w = w_ref[...]
    e = e_ref[...]
    e2 = jnp.sum(e * e, axis=1, keepdims=True)
    e_aug = jnp.concatenate([e, e2], axis=1)
    ones = jnp.ones((w.shape[0], 1), jnp.float32)
    w_aug = jnp.concatenate([-2.0 * w, ones], axis=1)
    scores = jax.lax.dot_general(
        e_aug, w_aug, (((1,), (1,)), ((), ())),
        preferred_element_type=jnp.float32,
        precision=jax.lax.Precision.HIGHEST)
    m = jnp.min(scores, axis=0, keepdims=True)
    ids = jax.lax.broadcasted_iota(jnp.int32, scores.shape, 0)
    idxv = jnp.min(
        jnp.where(scores == m, ids, _N_EMB), axis=0, keepdims=True)
    idx_ref[...] = idxv.reshape(1, 1, _BLK)
    part = jnp.sum(m) + jnp.sum(w * w)
    p_ref[0, 0] = part * (1.25 / (_N_TOK * _DIM))


def _tc_argmin(flat, embeddings):
    nblk = _N_TOK // _BLK
    return pl.pallas_call(
        _tc_body,
        grid=(nblk,),
        in_specs=[
            pl.BlockSpec((_BLK, _DIM), lambda i: (i, 0)),
            pl.BlockSpec((_N_EMB, _DIM), lambda i: (0, 0)),
        ],
        out_specs=[
            pl.BlockSpec((1, 1, _BLK), lambda i: (i, 0, 0)),
            pl.BlockSpec(memory_space=pltpu.SMEM),
        ],
        out_shape=[
            jax.ShapeDtypeStruct((_N_TOK // _BLK, 1, _BLK), jnp.int32),
            jax.ShapeDtypeStruct((1, 1), jnp.float32),
        ],
    )(flat, embeddings)


@functools.partial(
    pl.kernel,
    mesh=plsc.VectorSubcoreMesh(core_axis_name="c", subcore_axis_name="s"),
    out_type=jax.ShapeDtypeStruct((_N_TOK, _DIM), jnp.float32),
    scratch_types=[
        pltpu.VMEM((_CHUNKS, _IDX_CHUNK), jnp.int32),
        pltpu.VMEM((_ROWS_PER_W, _DIM), jnp.float32),
        pltpu.SemaphoreType.DMA,
    ],
    compiler_params=pltpu.CompilerParams(use_tc_tiling_on_sc=False),
)
def _sc_gather(table_hbm, idx_hbm, out_hbm, idx_v, rows_v, sem):
    wid = lax.axis_index("s") * _NC + lax.axis_index("c")
    base = wid * _ROWS_PER_W
    pltpu.sync_copy(idx_hbm.at[pl.ds(wid * _CHUNKS, _CHUNKS)], idx_v)
    copies = []
    for j in range(_CHUNKS):
        copies.append(pltpu.async_copy(
            table_hbm.at[idx_v.at[j]],
            rows_v.at[pl.ds(j * _IDX_CHUNK, _IDX_CHUNK)], sem))
    for c in copies:
        c.wait()
    pltpu.sync_copy(rows_v, out_hbm.at[pl.ds(base, _ROWS_PER_W)])


def kernel(weights, embeddings):
    orig_shape = weights.shape
    flat = weights.reshape(-1, _DIM)
    idx, loss = _tc_argmin(flat, embeddings)
    q = _sc_gather(embeddings, idx.reshape(_N_TOK // _IDX_CHUNK, _IDX_CHUNK))
    return q.reshape(orig_shape), loss.reshape(())
